# Initial kernel scaffold; baseline (speedup 1.0000x reference)
import jax
import jax.numpy as jnp
from jax.experimental import pallas as pl


def kernel(x, base0_w, base0_b, base1_w, base1_b, base2_w, base2_b, base3_w, base3_b, base4_w, base4_b, base5_w, base5_b, base6_w, base6_b, base7_w, base7_b, base8_w, base8_b, base9_w, base9_b, base10_w, base10_b, base11_w, base11_b, base12_w, base12_b, feat0_w, feat0_b, feat1_w, feat1_b, feat2_w, feat2_b, feat3_w, feat3_b, feat4_w, feat4_b, pool0_w, pool0_b, pool1_w, pool1_b, pool2_w, pool2_b, pool3_w, pool3_b, pool4_w, pool4_b, glob0_w, glob0_b, glob1_w, glob1_b, glob2_w, glob2_b, conv_g_w, conv_g_b, conv_l_w, conv_l_b):
    raise NotImplementedError("write your pallas kernel here")


# speedup vs baseline: 1.1725x; 1.1725x over previous
import jax
import jax.numpy as jnp
from jax.experimental import pallas as pl
from jax.experimental.pallas import tpu as pltpu


class _Act:
    def __init__(self, ref, H, W, stride, C, batched=False):
        self.ref, self.H, self.W, self.stride, self.C = ref, H, W, stride, C
        self.batched = batched

    def read(self, h):
        if self.batched:
            return self.ref[0, pl.ds(h * self.stride, self.W), :]
        return self.ref[pl.ds(h * self.stride, self.W), pl.ds(0, self.C)]


def _fill_canvas(cv, acts, ups, p, H, W, Wp, PAD_ROWS, Cin):
    if p > 0:
        head = p * Wp + p
        cv[pl.ds(0, head), pl.ds(0, Cin)] = jnp.zeros((head, Cin), jnp.bfloat16)
        side = jnp.zeros((2 * p, Cin), jnp.bfloat16)
        for h in range(H):
            cv[pl.ds((h + p) * Wp + p + W, 2 * p), pl.ds(0, Cin)] = side
        tail0 = (H + p) * Wp + p
        cv[pl.ds(tail0, PAD_ROWS - tail0), pl.ds(0, Cin)] = jnp.zeros(
            (PAD_ROWS - tail0, Cin), jnp.bfloat16)
    else:
        tail = PAD_ROWS - H * Wp
        cv[pl.ds(H * Wp, tail), pl.ds(0, Cin)] = jnp.zeros((tail, Cin),
                                                           jnp.bfloat16)
    off = 0
    for a, u in zip(acts, ups):
        ci = a.C
        if u == 1:
            for h in range(H):
                cv[pl.ds((h + p) * Wp + p, W), pl.ds(off, ci)] = (
                    a.read(h).astype(jnp.bfloat16))
        else:
            ri = jax.lax.broadcasted_iota(jnp.int32, (W, a.W), 0)
            cj = jax.lax.broadcasted_iota(jnp.int32, (W, a.W), 1)
            sel = jnp.where(ri // u == cj, 1.0, 0.0)
            for hs in range(a.H):
                row = jnp.dot(sel, a.read(hs),
                              preferred_element_type=jnp.float32)
                row = row.astype(jnp.bfloat16)
                for r in range(u):
                    cv[pl.ds((u * hs + r + p) * Wp + p, W),
                       pl.ds(off, ci)] = row
        off += ci


def _conv(acts, ups, w_ref, b_ref, out_ref, cv, im, *, p, KH, KW, O,
          relu):
    H, W = acts[0].H * ups[0], acts[0].W * ups[0]
    Cin = sum(a.C for a in acts)
    Hp, Wp = H + 2 * p, W + 2 * p
    OH, OW = Hp - KH + 1, Wp - KW + 1
    M = OH * Wp
    PAD_ROWS = Hp * Wp + (KW - 1)
    _fill_canvas(cv, acts, ups, p, H, W, Wp, PAD_ROWS, Cin)

    if im is not None:
        T = KH * KW
        for kh in range(KH):
            for kw in range(KW):
                im[pl.ds(0, M), pl.ds((kh * KW + kw) * Cin, Cin)] = (
                    cv[pl.ds(kh * Wp + kw, M), pl.ds(0, Cin)])
        res = jnp.dot(im[pl.ds(0, M), pl.ds(0, T * Cin)], w_ref[...],
                      preferred_element_type=jnp.float32) + b_ref[...]
        if relu:
            res = jnp.maximum(res, 0.0)
        out_ref[pl.ds(0, M), pl.ds(0, O)] = res
    else:
        first = True
        for kh in range(KH):
            for kw in range(KW):
                d = jnp.dot(cv[pl.ds(kh * Wp + kw, M), pl.ds(0, Cin)],
                            w_ref[kh, kw],
                            preferred_element_type=jnp.float32)
                if first:
                    out_ref[pl.ds(0, M), pl.ds(0, O)] = d + b_ref[...]
                    first = False
                else:
                    out_ref[pl.ds(0, M), pl.ds(0, O)] += d
        if relu:
            out_ref[pl.ds(0, M), pl.ds(0, O)] = jnp.maximum(
                out_ref[pl.ds(0, M), pl.ds(0, O)], 0.0)
    return _Act(out_ref, OH, OW, Wp, O)


def _contrast(f, c_ref, fp, O=64):
    OH, OW, Wp = f.H, f.W, f.stride
    M = OH * Wp
    FP_ROWS = (OH + 2) * Wp + 2
    fp[pl.ds(0, FP_ROWS), pl.ds(0, O)] = jnp.zeros((FP_ROWS, O), jnp.float32)
    for oh in range(OH):
        fp[pl.ds((oh + 1) * Wp + 1, OW), pl.ds(0, O)] = (
            f.ref[pl.ds(oh * Wp, OW), pl.ds(0, O)])
    s = fp[pl.ds(0, M), pl.ds(0, O)]
    for i in range(3):
        for j in range(3):
            if i == 0 and j == 0:
                continue
            s = s + fp[pl.ds(i * Wp + j, M), pl.ds(0, O)]
    c_ref[pl.ds(0, M), pl.ds(0, O)] = (
        f.ref[pl.ds(0, M), pl.ds(0, O)] - s * (1.0 / 9.0))
    return _Act(c_ref, OH, OW, Wp, O)


def _max_pool_2x2(a, out_ref):
    OH, OW, C = a.H // 2, a.W // 2, a.C
    ri = jax.lax.broadcasted_iota(jnp.int32, (OW, a.W), 0)
    cj = jax.lax.broadcasted_iota(jnp.int32, (OW, a.W), 1)
    sel_e = jnp.where(cj == 2 * ri, 1.0, 0.0)
    sel_o = jnp.where(cj == 2 * ri + 1, 1.0, 0.0)
    for oh in range(OH):
        r0 = a.read(2 * oh)
        r1 = a.read(2 * oh + 1)
        rm = jnp.maximum(r0, r1)
        e = jnp.dot(sel_e, rm, preferred_element_type=jnp.float32)
        o = jnp.dot(sel_o, rm, preferred_element_type=jnp.float32)
        out_ref[pl.ds(oh * OW, OW), pl.ds(0, C)] = jnp.maximum(e, o)
    return _Act(out_ref, OH, OW, OW, C)


def _max_pool_3x3(a, out_ref, pad_ref):
    H, W, C = a.H, a.W, a.C
    Wp = W + 2
    L = H * Wp
    PAD = (H + 2) * Wp + 2
    pad_ref[pl.ds(0, PAD), pl.ds(0, C)] = jnp.full((PAD, C), -3.0e38,
                                                   jnp.float32)
    for h in range(H):
        pad_ref[pl.ds((h + 1) * Wp + 1, W), pl.ds(0, C)] = a.read(h)
    m = pad_ref[pl.ds(0, L), pl.ds(0, C)]
    for i in range(3):
        for j in range(3):
            if i == 0 and j == 0:
                continue
            m = jnp.maximum(m, pad_ref[pl.ds(i * Wp + j, L), pl.ds(0, C)])
    out_ref[pl.ds(0, L), pl.ds(0, C)] = m
    return _Act(out_ref, H, W, Wp, C)


def kernel(x, base0_w, base0_b, base1_w, base1_b, base2_w, base2_b,
           base3_w, base3_b, base4_w, base4_b, base5_w, base5_b,
           base6_w, base6_b, base7_w, base7_b, base8_w, base8_b,
           base9_w, base9_b, base10_w, base10_b, base11_w, base11_b,
           base12_w, base12_b,
           feat0_w, feat0_b, feat1_w, feat1_b, feat2_w, feat2_b,
           feat3_w, feat3_b, feat4_w, feat4_b,
           pool0_w, pool0_b, pool1_w, pool1_b, pool2_w, pool2_b,
           pool3_w, pool3_b, pool4_w, pool4_b,
           glob0_w, glob0_b, glob1_w, glob1_b, glob2_w, glob2_b,
           conv_g_w, conv_g_b, conv_l_w, conv_l_b):
    N, _, H0, W0 = x.shape
    x_flat = jnp.transpose(x.astype(jnp.float32), (0, 2, 3, 1)).reshape(
        N, H0 * W0, 3)

    IM2COL_MAX = 128

    def prep(w, b, force_4d=False):
        KH, KW, Cin, O = w.shape
        wb = w.astype(jnp.bfloat16)
        if Cin <= IM2COL_MAX and not force_4d:
            wb = wb.reshape(KH * KW * Cin, O)
        return wb, b.reshape(1, O).astype(jnp.float32)

    base_w = [base0_w, base1_w, base2_w, base3_w, base4_w, base5_w, base6_w,
              base7_w, base8_w, base9_w, base10_w, base11_w, base12_w]
    base_b = [base0_b, base1_b, base2_b, base3_b, base4_b, base5_b, base6_b,
              base7_b, base8_b, base9_b, base10_b, base11_b, base12_b]
    feat_w = [feat0_w, feat1_w, feat2_w, feat3_w, feat4_w]
    feat_b = [feat0_b, feat1_b, feat2_b, feat3_b, feat4_b]
    pool_w = [pool0_w, pool1_w, pool2_w, pool3_w, pool4_w]
    pool_b = [pool0_b, pool1_b, pool2_b, pool3_b, pool4_b]
    glob_w = [glob0_w, glob1_w, glob2_w]
    glob_b = [glob0_b, glob1_b, glob2_b]

    operands = [x_flat]
    all_w = base_w + feat_w + pool_w + glob_w
    all_b = base_b + feat_b + pool_b + glob_b
    force_4d = [False] * 24 + [True, False]
    for w, b, f4 in zip(all_w, all_b, force_4d):
        wb, bb = prep(w, b, f4)
        operands.append(wb)
        operands.append(bb)
    operands.append(conv_l_w.reshape(1, 640).astype(jnp.float32))
    operands.append(conv_g_w.reshape(1, 128).astype(jnp.float32))
    operands.append((conv_l_b + conv_g_b).reshape(1, 1).astype(jnp.float32))

    in_specs = [pl.BlockSpec((1, H0 * W0, 3), lambda n: (n, 0, 0))]
    for op in operands[1:]:
        in_specs.append(
            pl.BlockSpec(op.shape, lambda n, nd=op.ndim: (0,) * nd))

    def body(*refs):
        x_ref = refs[0]
        wr = refs[1:53]
        wl_ref, wg_ref, sb_ref = refs[53], refs[54], refs[55]
        prob_ref = refs[56]
        (cvs, cvm, cvd, cvl, ims, imb, pool_pad, fpc,
         t44a, t44b, t22a, t22b, t11a, t11b,
         s0f, s0c, s1f, s1c, s2f, s2c, s3f, s3c, s4f, s4c,
         dec, decf, g7, g3) = refs[57:]

        bw = [(wr[2 * i], wr[2 * i + 1]) for i in range(13)]
        fw = [(wr[2 * i + 26], wr[2 * i + 27]) for i in range(5)]
        pw = [(wr[2 * i + 36], wr[2 * i + 37]) for i in range(5)]
        gw = [(wr[2 * i + 46], wr[2 * i + 47]) for i in range(3)]

        a = _Act(x_ref, 44, 44, 44, 3, batched=True)
        a = _conv([a], [1], *bw[0], t44a, cvs, ims, p=1, KH=3, KW=3, O=16,
                  relu=True)
        a = _conv([a], [1], *bw[1], t44b, cvs, ims, p=1, KH=3, KW=3, O=16,
                  relu=True)
        a = _max_pool_2x2(a, t22a)
        f0 = _conv([a], [1], *fw[0], s0f, cvm, imb, p=1, KH=3, KW=3, O=64,
                   relu=True)
        c0 = _contrast(f0, s0c, fpc)
        a = _conv([a], [1], *bw[2], t22b, cvm, imb, p=1, KH=3, KW=3, O=32,
                  relu=True)
        a = _conv([a], [1], *bw[3], t22a, cvm, imb, p=1, KH=3, KW=3, O=32,
                  relu=True)
        a = _max_pool_2x2(a, t11a)
        f1 = _conv([a], [1], *fw[1], s1f, cvm, imb, p=1, KH=3, KW=3, O=64,
                   relu=True)
        c1 = _contrast(f1, s1c, fpc)
        a = _conv([a], [1], *bw[4], t11b, cvm, imb, p=1, KH=3, KW=3, O=32,
                  relu=True)
        a = _conv([a], [1], *bw[5], t11a, cvm, imb, p=1, KH=3, KW=3, O=32,
                  relu=True)
        a = _conv([a], [1], *bw[6], t11b, cvm, imb, p=1, KH=3, KW=3, O=32,
                  relu=True)
        a = _max_pool_3x3(a, t11a, pool_pad)
        f2 = _conv([a], [1], *fw[2], s2f, cvm, imb, p=1, KH=3, KW=3, O=64,
                   relu=True)
        c2 = _contrast(f2, s2c, fpc)
        a = _conv([a], [1], *bw[7], t11b, cvm, imb, p=1, KH=3, KW=3, O=64,
                  relu=True)
        a = _conv([a], [1], *bw[8], t11a, cvm, imb, p=1, KH=3, KW=3, O=64,
                  relu=True)
        a = _conv([a], [1], *bw[9], t11b, cvm, imb, p=1, KH=3, KW=3, O=64,
                  relu=True)
        a = _max_pool_3x3(a, t11a, pool_pad)
        f3 = _conv([a], [1], *fw[3], s3f, cvm, imb, p=1, KH=3, KW=3, O=64,
                   relu=True)
        c3 = _contrast(f3, s3c, fpc)
        a = _conv([a], [1], *bw[10], t11b, cvm, imb, p=1, KH=3, KW=3, O=512,
                  relu=True)
        a = _conv([a], [1], *bw[11], t11a, cvl, None, p=1, KH=3, KW=3, O=512,
                  relu=True)
        a = _conv([a], [1], *bw[12], t11b, cvl, None, p=1, KH=3, KW=3, O=512,
                  relu=True)
        a = _max_pool_3x3(a, t11a, pool_pad)
        f4 = _conv([a], [1], *fw[4], s4f, cvl, None, p=1, KH=3, KW=3, O=64,
                   relu=True)
        c4 = _contrast(f4, s4c, fpc)

        g = _conv([a], [1], *gw[0], g7, cvl, None, p=0, KH=5, KW=5, O=128,
                  relu=True)
        g = _conv([g], [1], *gw[1], g3, cvl, None, p=0, KH=5, KW=5, O=128,
                  relu=True)
        g = _conv([g], [1], *gw[2], g7, cvl, imb, p=0, KH=3, KW=3, O=128,
                  relu=False)
        gs = jnp.sum(g7[pl.ds(0, 1), :] * wg_ref[...], axis=-1,
                     keepdims=True)

        d = _conv([f4, c4], [1, 1], *pw[4], dec, cvl, imb, p=1, KH=3, KW=3,
                  O=128, relu=True)
        d = _conv([f3, c3, d], [1, 1, 1], *pw[3], dec, cvl, None, p=1,
                  KH=3, KW=3, O=128, relu=True)
        d = _conv([f2, c2, d], [1, 1, 1], *pw[2], dec, cvl, None, p=1,
                  KH=3, KW=3, O=128, relu=True)
        d = _conv([f1, c1, d], [1, 1, 1], *pw[1], dec, cvl, None, p=1,
                  KH=3, KW=3, O=128, relu=True)
        d = _conv([f0, c0, d], [1, 1, 2], *pw[0], decf, cvd, None, p=1,
                  KH=3, KW=3, O=640, relu=False)

        M = 22 * 24
        s = jnp.sum(decf[pl.ds(0, M), :] * wl_ref[...], axis=-1,
                    keepdims=True)
        z = s + gs + sb_ref[...]
        prob_ref[0] = 1.0 / (1.0 + jnp.exp(-z))

    scratch = [
        pltpu.VMEM((2120, 16), jnp.bfloat16),
        pltpu.VMEM((584, 64), jnp.bfloat16),
        pltpu.VMEM((584, 256), jnp.bfloat16),
        pltpu.VMEM((176, 512), jnp.bfloat16),
        pltpu.VMEM((2024, 144), jnp.bfloat16),
        pltpu.VMEM((528, 1152), jnp.bfloat16),
        pltpu.VMEM((176, 512), jnp.float32),
        pltpu.VMEM((584, 64), jnp.float32),
        pltpu.VMEM((2024, 16), jnp.float32),
        pltpu.VMEM((2024, 16), jnp.float32),
        pltpu.VMEM((528, 32), jnp.float32),
        pltpu.VMEM((528, 32), jnp.float32),
        pltpu.VMEM((144, 512), jnp.float32),
        pltpu.VMEM((144, 512), jnp.float32),
        pltpu.VMEM((528, 64), jnp.float32),
        pltpu.VMEM((528, 64), jnp.float32),
        pltpu.VMEM((144, 64), jnp.float32),
        pltpu.VMEM((144, 64), jnp.float32),
        pltpu.VMEM((144, 64), jnp.float32),
        pltpu.VMEM((144, 64), jnp.float32),
        pltpu.VMEM((144, 64), jnp.float32),
        pltpu.VMEM((144, 64), jnp.float32),
        pltpu.VMEM((144, 64), jnp.float32),
        pltpu.VMEM((144, 64), jnp.float32),
        pltpu.VMEM((144, 128), jnp.float32),
        pltpu.VMEM((528, 640), jnp.float32),
        pltpu.VMEM((80, 128), jnp.float32),
        pltpu.VMEM((24, 128), jnp.float32),
    ]

    prob = pl.pallas_call(
        body,
        out_shape=jax.ShapeDtypeStruct((N, 528, 1), jnp.float32),
        grid=(N,),
        in_specs=in_specs,
        out_specs=pl.BlockSpec((1, 528, 1), lambda n: (n, 0, 0)),
        scratch_shapes=scratch,
        compiler_params=pltpu.CompilerParams(
            dimension_semantics=("parallel",),
            vmem_limit_bytes=100 * 1024 * 1024),
    )(*operands)

    prob = prob.reshape(N, 22, 24)[:, :, :22]
    return prob[:, None, :, :]


# speedup vs baseline: 1.1994x; 1.0229x over previous
import jax
import jax.numpy as jnp
from jax.experimental import pallas as pl
from jax.experimental.pallas import tpu as pltpu


_G44 = (44, 44, 48, 2210, 49, 2112)
_G22 = (22, 22, 24, 578, 25, 528)
_G11 = (11, 11, 13, 171, 14, 143)


def _taps(src, base, Wp, M, Cin, KH, KW, w_ref, b_ref, relu, batched=False,
          im=None):
    def slab(kh, kw):
        s = base + kh * Wp + kw
        if batched:
            return src[0, pl.ds(s, M), :]
        return src[pl.ds(s, M), pl.ds(0, Cin)]

    if im is not None:
        T = KH * KW
        for kh in range(KH):
            for kw in range(KW):
                im[pl.ds(0, M), pl.ds((kh * KW + kw) * Cin, Cin)] = (
                    slab(kh, kw))
        res = jnp.dot(im[pl.ds(0, M), pl.ds(0, T * Cin)], w_ref[...],
                      preferred_element_type=jnp.float32) + b_ref[...]
    else:
        res = None
        for kh in range(KH):
            for kw in range(KW):
                d = jnp.dot(slab(kh, kw), w_ref[kh, kw],
                            preferred_element_type=jnp.float32)
                res = (d + b_ref[...]) if res is None else (res + d)
    if relu:
        res = jnp.maximum(res, 0.0)
    return res


def _store(dst, val, geom, lane_off, C):
    off, M = geom[4], geom[5]
    dst[pl.ds(off, M), pl.ds(lane_off, C)] = val.astype(jnp.bfloat16)


def _halo(dst, geom, C):
    H, W, Wp, PAD, off, M = geom
    dst[pl.ds(0, Wp + 1), pl.ds(0, C)] = jnp.zeros((Wp + 1, C), jnp.bfloat16)
    side = jnp.zeros((Wp - W, C), jnp.bfloat16)
    for r in range(1, H + 1):
        dst[pl.ds(r * Wp + W + 1, Wp - W), pl.ds(0, C)] = side
    t0 = (H + 1) * Wp + 1
    dst[pl.ds(t0, PAD - t0), pl.ds(0, C)] = jnp.zeros((PAD - t0, C),
                                                      jnp.bfloat16)


def _conv_to(src, sgeom, dst, dgeom, Cin, w_ref, b_ref, relu,
             batched=False, lane_off=0, O=None, im=None):
    res = _taps(src, 0, sgeom[2], sgeom[5], Cin, 3, 3, w_ref, b_ref, relu,
                batched=batched, im=im)
    _store(dst, res, dgeom, lane_off, O)
    return res


def _max_pool_2x2(src, sgeom, dst, dgeom, C):
    H, W, Wp = sgeom[0], sgeom[1], sgeom[2]
    OH, OW = H // 2, W // 2
    ri = jax.lax.broadcasted_iota(jnp.int32, (OW, W), 0)
    cj = jax.lax.broadcasted_iota(jnp.int32, (OW, W), 1)
    sel_e = jnp.where(cj == 2 * ri, 1.0, 0.0).astype(jnp.bfloat16)
    sel_o = jnp.where(cj == 2 * ri + 1, 1.0, 0.0).astype(jnp.bfloat16)
    dWp = dgeom[2]
    for oh in range(OH):
        r0 = src[pl.ds((2 * oh + 1) * Wp + 1, W), pl.ds(0, C)]
        r1 = src[pl.ds((2 * oh + 2) * Wp + 1, W), pl.ds(0, C)]
        rm = jnp.maximum(r0, r1)
        e = jnp.dot(sel_e, rm, preferred_element_type=jnp.float32)
        o = jnp.dot(sel_o, rm, preferred_element_type=jnp.float32)
        dst[pl.ds((oh + 1) * dWp + 1, OW), pl.ds(0, C)] = jnp.maximum(
            e, o).astype(jnp.bfloat16)


def _max_pool_3x3(src, dst, geom, C):
    Wp, off, M = geom[2], geom[4], geom[5]
    m = None
    for i in range(3):
        for j in range(3):
            sl = src[pl.ds(i * Wp + j, M), pl.ds(0, C)]
            m = sl if m is None else jnp.maximum(m, sl)
    dst[pl.ds(off, M), pl.ds(0, C)] = m


def _feat(src, sgeom, w_ref, b_ref, dst, fpc, Cin, im=None):
    H, W, Wp, PAD, off, M = sgeom
    f = _taps(src, 0, Wp, M, Cin, 3, 3, w_ref, b_ref, True, im=im)
    _store(dst, f, sgeom, 0, 64)
    fpc[pl.ds(0, PAD), pl.ds(0, 64)] = jnp.zeros((PAD, 64), jnp.float32)
    for oh in range(H):
        fpc[pl.ds((oh + 1) * Wp + 1, W), pl.ds(0, 64)] = f[oh * Wp:oh * Wp + W]
    s = None
    for i in range(3):
        for j in range(3):
            sl = fpc[pl.ds(i * Wp + j, M), pl.ds(0, 64)]
            s = sl if s is None else s + sl
    _store(dst, f - s * (1.0 / 9.0), sgeom, 64, 64)


def kernel(x, base0_w, base0_b, base1_w, base1_b, base2_w, base2_b,
           base3_w, base3_b, base4_w, base4_b, base5_w, base5_b,
           base6_w, base6_b, base7_w, base7_b, base8_w, base8_b,
           base9_w, base9_b, base10_w, base10_b, base11_w, base11_b,
           base12_w, base12_b,
           feat0_w, feat0_b, feat1_w, feat1_b, feat2_w, feat2_b,
           feat3_w, feat3_b, feat4_w, feat4_b,
           pool0_w, pool0_b, pool1_w, pool1_b, pool2_w, pool2_b,
           pool3_w, pool3_b, pool4_w, pool4_b,
           glob0_w, glob0_b, glob1_w, glob1_b, glob2_w, glob2_b,
           conv_g_w, conv_g_b, conv_l_w, conv_l_b):
    N = x.shape[0]
    x_nhwc = jnp.transpose(x.astype(jnp.float32), (0, 2, 3, 1))
    x_pad = jnp.pad(x_nhwc, ((0, 0), (1, 1), (1, 3), (0, 0)))
    x_pad = x_pad.reshape(N, 46 * 48, 3)
    x_pad = jnp.pad(x_pad, ((0, 0), (0, 2), (0, 0))).astype(jnp.bfloat16)

    base_w = [base0_w, base1_w, base2_w, base3_w, base4_w, base5_w, base6_w,
              base7_w, base8_w, base9_w, base10_w, base11_w, base12_w]
    base_b = [base0_b, base1_b, base2_b, base3_b, base4_b, base5_b, base6_b,
              base7_b, base8_b, base9_b, base10_b, base11_b, base12_b]
    feat_w = [feat0_w, feat1_w, feat2_w, feat3_w, feat4_w]
    feat_b = [feat0_b, feat1_b, feat2_b, feat3_b, feat4_b]
    pool_w = [pool0_w, pool1_w, pool2_w, pool3_w, pool4_w]
    pool_b = [pool0_b, pool1_b, pool2_b, pool3_b, pool4_b]
    glob_w = [glob0_w, glob1_w, glob2_w]
    glob_b = [glob0_b, glob1_b, glob2_b]

    operands = [x_pad]
    for w, b in zip(base_w + feat_w + pool_w + glob_w,
                    base_b + feat_b + pool_b + glob_b):
        KH, KW, Cin, O = w.shape
        wb = w.astype(jnp.bfloat16)
        if Cin <= 128:
            wb = wb.reshape(KH * KW * Cin, O)
        operands.append(wb)
        operands.append(b.reshape(1, -1).astype(jnp.float32))
    operands.append(conv_l_w.reshape(1, 640).astype(jnp.float32))
    operands.append(conv_g_w.reshape(1, 128).astype(jnp.float32))
    operands.append((conv_l_b + conv_g_b).reshape(1, 1).astype(jnp.float32))

    in_specs = [pl.BlockSpec((1, 2210, 3), lambda n: (n, 0, 0))]
    for op in operands[1:]:
        in_specs.append(
            pl.BlockSpec(op.shape, lambda n, nd=op.ndim: (0,) * nd))

    def body(*refs):
        x_ref = refs[0]
        wr = refs[1:53]
        wl_ref, wg_ref, sb_ref = refs[53], refs[54], refs[55]
        prob_ref = refs[56]
        (c44a, c44b, c22a, c22b, c11a, c11b,
         dc0, dc1, dc2, dc3, dc4, gb1, gb2, fpc,
         im44, imm, img) = refs[57:]

        bw = [(wr[2 * i], wr[2 * i + 1]) for i in range(13)]
        fw = [(wr[2 * i + 26], wr[2 * i + 27]) for i in range(5)]
        pw = [(wr[2 * i + 36], wr[2 * i + 37]) for i in range(5)]
        gw = [(wr[2 * i + 46], wr[2 * i + 47]) for i in range(3)]

        _conv_to(x_ref, _G44, c44a, _G44, 3, *bw[0], True, batched=True,
                 O=16, im=im44)
        _halo(c44a, _G44, 16)
        _conv_to(c44a, _G44, c44b, _G44, 16, *bw[1], True, O=16, im=im44)
        _halo(c44b, _G44, 16)
        _max_pool_2x2(c44b, _G44, c22a, _G22, 16)
        _halo(c22a, _G22, 16)
        _feat(c22a, _G22, *fw[0], dc0, fpc, 16, im=imm)
        _conv_to(c22a, _G22, c22b, _G22, 16, *bw[2], True, O=32, im=imm)
        _halo(c22b, _G22, 32)
        _conv_to(c22b, _G22, c22a, _G22, 32, *bw[3], True, O=32, im=imm)
        _halo(c22a, _G22, 32)
        _max_pool_2x2(c22a, _G22, c11a, _G11, 32)
        _halo(c11a, _G11, 32)
        _feat(c11a, _G11, *fw[1], dc1, fpc, 32, im=imm)
        _conv_to(c11a, _G11, c11b, _G11, 32, *bw[4], True, O=32, im=imm)
        _halo(c11b, _G11, 32)
        _conv_to(c11b, _G11, c11a, _G11, 32, *bw[5], True, O=32, im=imm)
        _halo(c11a, _G11, 32)
        _conv_to(c11a, _G11, c11b, _G11, 32, *bw[6], True, O=32, im=imm)
        _halo(c11b, _G11, 32)
        _max_pool_3x3(c11b, c11a, _G11, 32)
        _halo(c11a, _G11, 32)
        _feat(c11a, _G11, *fw[2], dc2, fpc, 32, im=imm)
        _conv_to(c11a, _G11, c11b, _G11, 32, *bw[7], True, O=64, im=imm)
        _halo(c11b, _G11, 64)
        _conv_to(c11b, _G11, c11a, _G11, 64, *bw[8], True, O=64, im=imm)
        _halo(c11a, _G11, 64)
        _conv_to(c11a, _G11, c11b, _G11, 64, *bw[9], True, O=64, im=imm)
        _halo(c11b, _G11, 64)
        _max_pool_3x3(c11b, c11a, _G11, 64)
        _halo(c11a, _G11, 64)
        _feat(c11a, _G11, *fw[3], dc3, fpc, 64, im=imm)
        _conv_to(c11a, _G11, c11b, _G11, 64, *bw[10], True, O=512, im=imm)
        _halo(c11b, _G11, 512)
        _conv_to(c11b, _G11, c11a, _G11, 512, *bw[11], True, O=512)
        _halo(c11a, _G11, 512)
        _conv_to(c11a, _G11, c11b, _G11, 512, *bw[12], True, O=512)
        _halo(c11b, _G11, 512)
        _max_pool_3x3(c11b, c11a, _G11, 512)
        _halo(c11a, _G11, 512)
        _feat(c11a, _G11, *fw[4], dc4, fpc, 512)

        g0 = _taps(c11a, 14, 13, 85, 512, 5, 5, *gw[0], True)
        gb1[pl.ds(0, 85), :] = g0.astype(jnp.bfloat16)
        g1 = _taps(gb1, 0, 13, 29, 128, 5, 5, *gw[1], True,
                   im=img)
        gb2[pl.ds(0, 29), :] = g1.astype(jnp.bfloat16)
        g2 = _taps(gb2, 0, 13, 1, 128, 3, 3, *gw[2], False,
                   im=img)
        gs = jnp.sum(g2 * wg_ref[...], axis=-1, keepdims=True)

        _halo(dc4, _G11, 128)
        d = _taps(dc4, 0, 13, 143, 128, 3, 3, *pw[4], True, im=imm)
        for k, dck in ((3, dc3), (2, dc2), (1, dc1)):
            _store(dck, d, _G11, 128, 128)
            _halo(dck, _G11, 256)
            d = _taps(dck, 0, 13, 143, 256, 3, 3, *pw[k], True)
        ri = jax.lax.broadcasted_iota(jnp.int32, (22, 11), 0)
        cj = jax.lax.broadcasted_iota(jnp.int32, (22, 11), 1)
        sel = jnp.where(ri // 2 == cj, 1.0, 0.0)
        for hs in range(11):
            row = jnp.dot(sel, d[hs * 13:hs * 13 + 11],
                          preferred_element_type=jnp.float32)
            row = row.astype(jnp.bfloat16)
            for r in range(2):
                dc0[pl.ds((2 * hs + r + 1) * 24 + 1, 22),
                    pl.ds(128, 128)] = row
        _halo(dc0, _G22, 256)
        out = _taps(dc0, 0, 24, 528, 256, 3, 3, *pw[0], False)

        s = jnp.sum(out * wl_ref[...], axis=-1, keepdims=True)
        z = s + gs + sb_ref[...]
        prob_ref[0] = 1.0 / (1.0 + jnp.exp(-z))

    scratch = [
        pltpu.VMEM((2216, 16), jnp.bfloat16),
        pltpu.VMEM((2216, 16), jnp.bfloat16),
        pltpu.VMEM((584, 32), jnp.bfloat16),
        pltpu.VMEM((584, 32), jnp.bfloat16),
        pltpu.VMEM((176, 512), jnp.bfloat16),
        pltpu.VMEM((176, 512), jnp.bfloat16),
        pltpu.VMEM((584, 256), jnp.bfloat16),
        pltpu.VMEM((176, 256), jnp.bfloat16),
        pltpu.VMEM((176, 256), jnp.bfloat16),
        pltpu.VMEM((176, 256), jnp.bfloat16),
        pltpu.VMEM((176, 256), jnp.bfloat16),
        pltpu.VMEM((88, 128), jnp.bfloat16),
        pltpu.VMEM((32, 128), jnp.bfloat16),
        pltpu.VMEM((584, 64), jnp.float32),
        pltpu.VMEM((2112, 144), jnp.bfloat16),
        pltpu.VMEM((528, 1152), jnp.bfloat16),
        pltpu.VMEM((32, 3200), jnp.bfloat16),
    ]

    prob = pl.pallas_call(
        body,
        out_shape=jax.ShapeDtypeStruct((N, 528, 1), jnp.float32),
        grid=(N,),
        in_specs=in_specs,
        out_specs=pl.BlockSpec((1, 528, 1), lambda n: (n, 0, 0)),
        scratch_shapes=scratch,
        compiler_params=pltpu.CompilerParams(
            dimension_semantics=("parallel",),
            vmem_limit_bytes=100 * 1024 * 1024),
    )(*operands)

    prob = prob.reshape(N, 22, 24)[:, :, :22]
    return prob[:, None, :, :]
